# baseline (speedup 1.0000x reference)
import functools

import jax
import jax.numpy as jnp
from jax import lax
from jax.experimental import pallas as pl
from jax.experimental.pallas import tpu as pltpu
from jax.experimental.pallas import tpu_sc as plsc

N_NODES = 10000
NP = 10240
E = 320000
D = 128
G = 64
NC = 2
NS = 16
NW = NC * NS
EPW = E // NW
C = 125
NCH = EPW // C
RPS = NP // NS
BLK = 1024
NBLK = NP // BLK

_mesh = plsc.VectorSubcoreMesh(core_axis_name="c", subcore_axis_name="s")


@functools.partial(
    pl.kernel,
    out_type=jax.ShapeDtypeStruct((NW, NP // 16, 16), jnp.float32),
    mesh=_mesh,
    scratch_types=[
        pltpu.VMEM((EPW,), jnp.int32),
        pltpu.VMEM((NP // 16, 16), jnp.float32),
    ],
    compiler_params=pltpu.CompilerParams(needs_layout_passes=False),
)
def _sc_degree(dst_hbm, out_hbm, didx, hist):
    cid = lax.axis_index("c")
    sid = lax.axis_index("s")
    wid = cid * NS + sid
    pltpu.sync_copy(dst_hbm.at[wid], didx)

    zeros = jnp.zeros((16,), jnp.float32)

    def zinit(i, carry):
        hist[i, :] = zeros
        return carry

    lax.fori_loop(0, NP // 16, zinit, 0)

    ones = jnp.ones((16,), jnp.float32)

    def body(i, carry):
        idx = didx[pl.ds(i * 16, 16)]
        plsc.addupdate_scatter(hist, [idx >> 4, idx & 15], ones)
        return carry

    lax.fori_loop(0, EPW // 16, body, 0)
    pltpu.sync_copy(hist, out_hbm.at[wid])


@functools.partial(
    pl.kernel,
    out_type=jax.ShapeDtypeStruct((NC, NP, D), jnp.float32),
    mesh=_mesh,
    scratch_types=[
        pltpu.VMEM((NCH, C), jnp.int32),
        pltpu.VMEM((NCH, C), jnp.int32),
        pltpu.VMEM((C, D), jnp.float32),
        pltpu.VMEM((C, D), jnp.float32),
        pltpu.VMEM_SHARED((NP, D), jnp.float32),
    ],
    compiler_params=pltpu.CompilerParams(needs_layout_passes=False),
)
def _sc_edge_agg(hs_hbm, src_hbm, dst_hbm, zeros_hbm, out_hbm,
                 sidx, didx, rows0, rows1, acc):
    cid = lax.axis_index("c")
    sid = lax.axis_index("s")
    wid = cid * NS + sid
    pltpu.sync_copy(src_hbm.at[wid], sidx)
    pltpu.sync_copy(dst_hbm.at[wid], didx)
    pltpu.sync_copy(zeros_hbm, acc.at[pl.ds(sid * RPS, RPS)])
    plsc.subcore_barrier()

    def body(j, carry):
        pltpu.sync_copy(hs_hbm.at[sidx.at[j]], rows0)
        pltpu.sync_copy(rows0, acc.at[didx.at[j]], add=True)
        return carry

    lax.fori_loop(0, NCH, body, 0)
    plsc.subcore_barrier()
    pltpu.sync_copy(acc.at[pl.ds(sid * RPS, RPS)], out_hbm.at[cid].at[pl.ds(sid * RPS, RPS)])


def _mm_body(x_ref, w_ref, h_ref):
    h_ref[...] = jnp.dot(x_ref[...], w_ref[...], preferred_element_type=jnp.float32)


def _tc_matmul(x_pad, W1):
    return pl.pallas_call(
        _mm_body,
        grid=(NBLK,),
        in_specs=[
            pl.BlockSpec((BLK, D), lambda i: (i, 0)),
            pl.BlockSpec((D, D), lambda i: (0, 0)),
        ],
        out_specs=pl.BlockSpec((BLK, D), lambda i: (i, 0)),
        out_shape=jax.ShapeDtypeStruct((NP, D), jnp.float32),
    )(x_pad, W1)


def _scale_body(deg_ref, h_ref, hs_ref, dinv_ref):
    total = jnp.sum(deg_ref[...], axis=1, keepdims=True) + 1.0
    dinv = lax.rsqrt(total)
    hs_ref[...] = h_ref[...] * dinv
    dinv_ref[...] = dinv


def _tc_scale(deg_t, h):
    return pl.pallas_call(
        _scale_body,
        grid=(NBLK,),
        in_specs=[
            pl.BlockSpec((BLK, NW), lambda i: (i, 0)),
            pl.BlockSpec((BLK, D), lambda i: (i, 0)),
        ],
        out_specs=[
            pl.BlockSpec((BLK, D), lambda i: (i, 0)),
            pl.BlockSpec((BLK, 1), lambda i: (i, 0)),
        ],
        out_shape=[
            jax.ShapeDtypeStruct((NP, D), jnp.float32),
            jax.ShapeDtypeStruct((NP, 1), jnp.float32),
        ],
    )(deg_t, h)


def _head_body(accA, accB, hs, dinv, brow, b1r, fcw, fcbr, out, sums, counts):
    i = pl.program_id(0)

    @pl.when(i == 0)
    def _():
        sums[...] = jnp.zeros_like(sums)
        counts[...] = jnp.zeros_like(counts)

    r = dinv[...] * (accA[...] + accB[...] + hs[...]) + b1r[...]
    r = jnp.maximum(r, 0.0)
    iota = lax.broadcasted_iota(jnp.int32, (G, BLK), 0)
    oh = (jnp.broadcast_to(brow[...], (G, BLK)) == iota).astype(jnp.float32)
    sums[...] += jnp.dot(oh, r, preferred_element_type=jnp.float32)
    counts[...] += jnp.sum(oh, axis=1, keepdims=True)

    @pl.when(i == NBLK - 1)
    def _():
        g = sums[...] / jnp.maximum(counts[...], 1.0)
        logits = jnp.dot(g, fcw[...], preferred_element_type=jnp.float32) + fcbr[...]
        m = jnp.max(logits, axis=1, keepdims=True)
        lse = m + jnp.log(jnp.sum(jnp.exp(logits - m), axis=1, keepdims=True))
        out[...] = logits - lse


def _tc_head(accA, accB, hs, dinv, batch_row, b1r, fcW, fcbr):
    return pl.pallas_call(
        _head_body,
        grid=(NBLK,),
        in_specs=[
            pl.BlockSpec((BLK, D), lambda i: (i, 0)),
            pl.BlockSpec((BLK, D), lambda i: (i, 0)),
            pl.BlockSpec((BLK, D), lambda i: (i, 0)),
            pl.BlockSpec((BLK, 1), lambda i: (i, 0)),
            pl.BlockSpec((1, BLK), lambda i: (0, i)),
            pl.BlockSpec((1, D), lambda i: (0, 0)),
            pl.BlockSpec((D, 2), lambda i: (0, 0)),
            pl.BlockSpec((1, 2), lambda i: (0, 0)),
        ],
        out_specs=pl.BlockSpec((G, 2), lambda i: (0, 0)),
        out_shape=jax.ShapeDtypeStruct((G, 2), jnp.float32),
        scratch_shapes=[
            pltpu.VMEM((G, D), jnp.float32),
            pltpu.VMEM((G, 1), jnp.float32),
        ],
        compiler_params=pltpu.CompilerParams(
            dimension_semantics=("arbitrary",),
        ),
    )(accA, accB, hs, dinv, batch_row, b1r, fcW, fcbr)


def kernel(x, edge_index, batch, W1, b1, fcW, fcb):
    src = edge_index[0].astype(jnp.int32)
    dst = edge_index[1].astype(jnp.int32)
    dst_w = dst.reshape(NW, EPW)
    src_ch = src.reshape(NW, NCH, C)
    dst_ch = dst.reshape(NW, NCH, C)
    x_pad = jnp.pad(x, ((0, NP - N_NODES), (0, 0)))
    batch_row = jnp.pad(
        batch.astype(jnp.int32), (0, NP - N_NODES), constant_values=G
    ).reshape(1, NP)
    zeros_nd = jnp.zeros((RPS, D), jnp.float32)

    deg_part = _sc_degree(dst_w)
    h = _tc_matmul(x_pad, W1)
    deg_t = deg_part.reshape(NW, NP).T
    hs, dinv = _tc_scale(deg_t, h)
    acc = _sc_edge_agg(hs, src_ch, dst_ch, zeros_nd)
    out = _tc_head(
        acc[0], acc[1], hs, dinv, batch_row,
        b1.reshape(1, D), fcW, fcb.reshape(1, 2),
    )
    return out


# speedup vs baseline: 1.0326x; 1.0326x over previous
import functools

import jax
import jax.numpy as jnp
from jax import lax
from jax.experimental import pallas as pl
from jax.experimental.pallas import tpu as pltpu
from jax.experimental.pallas import tpu_sc as plsc

N_NODES = 10000
NP = 10240
E = 320000
D = 128
G = 64
NC = 2
NS = 16
NW = NC * NS
EPW = E // NW
C = 125
NCH = EPW // C
RPS = NP // NS
BLK = 1024
NBLK = NP // BLK

_mesh = plsc.VectorSubcoreMesh(core_axis_name="c", subcore_axis_name="s")


@functools.partial(
    pl.kernel,
    out_type=jax.ShapeDtypeStruct((NW, NP // 16, 16), jnp.float32),
    mesh=_mesh,
    scratch_types=[
        pltpu.VMEM((EPW,), jnp.int32),
        pltpu.VMEM((NP // 16, 16), jnp.float32),
    ],
    compiler_params=pltpu.CompilerParams(needs_layout_passes=False),
)
def _sc_degree(dst_hbm, out_hbm, didx, hist):
    cid = lax.axis_index("c")
    sid = lax.axis_index("s")
    wid = cid * NS + sid
    pltpu.sync_copy(dst_hbm.at[wid], didx)

    zeros = jnp.zeros((16,), jnp.float32)

    def zinit(i, carry):
        hist[i, :] = zeros
        return carry

    lax.fori_loop(0, NP // 16, zinit, 0)

    ones = jnp.ones((16,), jnp.float32)

    def body(i, carry):
        idx = didx[pl.ds(i * 16, 16)]
        plsc.addupdate_scatter(hist, [idx >> 4, idx & 15], ones)
        return carry

    lax.fori_loop(0, EPW // 16, body, 0)
    pltpu.sync_copy(hist, out_hbm.at[wid])


@functools.partial(
    pl.kernel,
    out_type=jax.ShapeDtypeStruct((NC, NP, D), jnp.float32),
    mesh=_mesh,
    scratch_types=[
        pltpu.VMEM((NCH, C), jnp.int32),
        pltpu.VMEM((NCH, C), jnp.int32),
        pltpu.VMEM((C, D), jnp.float32),
        pltpu.VMEM((C, D), jnp.float32),
        pltpu.VMEM_SHARED((NP, D), jnp.float32),
    ],
    compiler_params=pltpu.CompilerParams(needs_layout_passes=False),
)
def _sc_edge_agg(hs_hbm, src_hbm, dst_hbm, zeros_hbm, out_hbm,
                 sidx, didx, rows0, rows1, acc):
    cid = lax.axis_index("c")
    sid = lax.axis_index("s")
    wid = cid * NS + sid
    pltpu.sync_copy(src_hbm.at[wid], sidx)
    pltpu.sync_copy(dst_hbm.at[wid], didx)
    pltpu.sync_copy(zeros_hbm, acc.at[pl.ds(sid * RPS, RPS)])
    plsc.subcore_barrier()

    def body(j, carry):
        pltpu.sync_copy(hs_hbm.at[sidx.at[j]], rows0)
        pltpu.sync_copy(rows0, acc.at[didx.at[j]], add=True)
        return carry

    lax.fori_loop(0, NCH, body, 0)
    plsc.subcore_barrier()
    pltpu.sync_copy(acc.at[pl.ds(sid * RPS, RPS)], out_hbm.at[cid].at[pl.ds(sid * RPS, RPS)])


def _mm_body(deg_ref, x_ref, w_ref, hs_ref, dinv_ref):
    total = jnp.sum(deg_ref[...], axis=1, keepdims=True) + 1.0
    dinv = lax.rsqrt(total)
    h = jnp.dot(x_ref[...], w_ref[...], preferred_element_type=jnp.float32)
    hs_ref[...] = h * dinv
    dinv_ref[...] = dinv


def _tc_matmul(deg_t, x_pad, W1):
    return pl.pallas_call(
        _mm_body,
        grid=(NBLK,),
        in_specs=[
            pl.BlockSpec((BLK, NW), lambda i: (i, 0)),
            pl.BlockSpec((BLK, D), lambda i: (i, 0)),
            pl.BlockSpec((D, D), lambda i: (0, 0)),
        ],
        out_specs=[
            pl.BlockSpec((BLK, D), lambda i: (i, 0)),
            pl.BlockSpec((BLK, 1), lambda i: (i, 0)),
        ],
        out_shape=[
            jax.ShapeDtypeStruct((NP, D), jnp.float32),
            jax.ShapeDtypeStruct((NP, 1), jnp.float32),
        ],
    )(deg_t, x_pad, W1)


def _head_body(accA, accB, hs, dinv, brow, b1r, fcw, fcbr, out, sums, counts):
    i = pl.program_id(0)

    @pl.when(i == 0)
    def _():
        sums[...] = jnp.zeros_like(sums)
        counts[...] = jnp.zeros_like(counts)

    r = dinv[...] * (accA[...] + accB[...] + hs[...]) + b1r[...]
    r = jnp.maximum(r, 0.0)
    iota = lax.broadcasted_iota(jnp.int32, (G, BLK), 0)
    oh = (jnp.broadcast_to(brow[...], (G, BLK)) == iota).astype(jnp.float32)
    sums[...] += jnp.dot(oh, r, preferred_element_type=jnp.float32)
    counts[...] += jnp.sum(oh, axis=1, keepdims=True)

    @pl.when(i == NBLK - 1)
    def _():
        g = sums[...] / jnp.maximum(counts[...], 1.0)
        logits = jnp.dot(g, fcw[...], preferred_element_type=jnp.float32) + fcbr[...]
        m = jnp.max(logits, axis=1, keepdims=True)
        lse = m + jnp.log(jnp.sum(jnp.exp(logits - m), axis=1, keepdims=True))
        out[...] = logits - lse


def _tc_head(accA, accB, hs, dinv, batch_row, b1r, fcW, fcbr):
    return pl.pallas_call(
        _head_body,
        grid=(NBLK,),
        in_specs=[
            pl.BlockSpec((BLK, D), lambda i: (i, 0)),
            pl.BlockSpec((BLK, D), lambda i: (i, 0)),
            pl.BlockSpec((BLK, D), lambda i: (i, 0)),
            pl.BlockSpec((BLK, 1), lambda i: (i, 0)),
            pl.BlockSpec((1, BLK), lambda i: (0, i)),
            pl.BlockSpec((1, D), lambda i: (0, 0)),
            pl.BlockSpec((D, 2), lambda i: (0, 0)),
            pl.BlockSpec((1, 2), lambda i: (0, 0)),
        ],
        out_specs=pl.BlockSpec((G, 2), lambda i: (0, 0)),
        out_shape=jax.ShapeDtypeStruct((G, 2), jnp.float32),
        scratch_shapes=[
            pltpu.VMEM((G, D), jnp.float32),
            pltpu.VMEM((G, 1), jnp.float32),
        ],
        compiler_params=pltpu.CompilerParams(
            dimension_semantics=("arbitrary",),
        ),
    )(accA, accB, hs, dinv, batch_row, b1r, fcW, fcbr)


def kernel(x, edge_index, batch, W1, b1, fcW, fcb):
    src = edge_index[0].astype(jnp.int32)
    dst = edge_index[1].astype(jnp.int32)
    dst_w = dst.reshape(NW, EPW)
    src_ch = src.reshape(NW, NCH, C)
    dst_ch = dst.reshape(NW, NCH, C)
    x_pad = jnp.pad(x, ((0, NP - N_NODES), (0, 0)))
    batch_row = jnp.pad(
        batch.astype(jnp.int32), (0, NP - N_NODES), constant_values=G
    ).reshape(1, NP)
    zeros_nd = jnp.zeros((RPS, D), jnp.float32)

    deg_part = _sc_degree(dst_w)
    deg_t = deg_part.reshape(NW, NP).T
    hs, dinv = _tc_matmul(deg_t, x_pad, W1)
    acc = _sc_edge_agg(hs, src_ch, dst_ch, zeros_nd)
    out = _tc_head(
        acc[0], acc[1], hs, dinv, batch_row,
        b1.reshape(1, D), fcW, fcb.reshape(1, 2),
    )
    return out


# speedup vs baseline: 1.1680x; 1.1312x over previous
import functools

import jax
import jax.numpy as jnp
from jax import lax
from jax.experimental import pallas as pl
from jax.experimental.pallas import tpu as pltpu
from jax.experimental.pallas import tpu_sc as plsc

N_NODES = 10000
NP = 10240
E = 320000
D = 128
G = 64
NC = 2
NS = 16
NW = NC * NS
EPW = E // NW
C = 125
NCH = EPW // C
RPS = NP // NS
BLK = 1024
NBLK = NP // BLK

_mesh = plsc.VectorSubcoreMesh(core_axis_name="c", subcore_axis_name="s")


@functools.partial(
    pl.kernel,
    out_type=jax.ShapeDtypeStruct((NW, NP // 16, 16), jnp.float32),
    mesh=_mesh,
    scratch_types=[
        pltpu.VMEM((EPW,), jnp.int32),
        pltpu.VMEM((NP // 16, 16), jnp.float32),
    ],
    compiler_params=pltpu.CompilerParams(needs_layout_passes=False),
)
def _sc_degree(dst_hbm, out_hbm, didx, hist):
    cid = lax.axis_index("c")
    sid = lax.axis_index("s")
    wid = cid * NS + sid
    pltpu.sync_copy(dst_hbm.at[wid], didx)

    zeros = jnp.zeros((16,), jnp.float32)

    def zinit(i, carry):
        hist[i, :] = zeros
        return carry

    lax.fori_loop(0, NP // 16, zinit, 0)

    ones = jnp.ones((16,), jnp.float32)

    def body(i, carry):
        idx = didx[pl.ds(i * 16, 16)]
        plsc.addupdate_scatter(hist, [idx >> 4, idx & 15], ones)
        return carry

    lax.fori_loop(0, EPW // 16, body, 0)
    pltpu.sync_copy(hist, out_hbm.at[wid])


@functools.partial(
    pl.kernel,
    out_type=jax.ShapeDtypeStruct((NC, NP, D), jnp.float32),
    mesh=_mesh,
    scratch_types=[
        pltpu.VMEM((8, C), jnp.int32),
        pltpu.VMEM((8, C), jnp.int32),
        pltpu.VMEM((C, D), jnp.float32),
        pltpu.VMEM((C, D), jnp.float32),
        pltpu.VMEM_SHARED((NP, D), jnp.float32),
    ],
    compiler_params=pltpu.CompilerParams(needs_layout_passes=False),
)
def _sc_edge_agg(hs_hbm, src_hbm, dst_hbm, zeros_hbm, out_hbm,
                 sblk, dblk, rows0, rows1, acc):
    cid = lax.axis_index("c")
    sid = lax.axis_index("s")
    wid = cid * NS + sid
    pltpu.sync_copy(zeros_hbm, acc.at[pl.ds(sid * RPS, RPS)])
    plsc.subcore_barrier()

    def with_sems(sem0, sem1):
        def outer(b, carry):
            pltpu.sync_copy(src_hbm.at[wid].at[b], sblk)
            pltpu.sync_copy(dst_hbm.at[wid].at[b], dblk)
            pltpu.sync_copy(hs_hbm.at[sblk.at[0]], rows0)

            def inner(jj, c2):
                l0 = jj * 2
                pltpu.async_copy(hs_hbm.at[sblk.at[l0 + 1]], rows1, sem1)
                pltpu.sync_copy(rows0, acc.at[dblk.at[l0]], add=True)
                pltpu.make_async_copy(hs_hbm.at[sblk.at[l0 + 1]], rows1, sem1).wait()

                @pl.when(jj < 3)
                def _():
                    pltpu.async_copy(hs_hbm.at[sblk.at[l0 + 2]], rows0, sem0)

                pltpu.sync_copy(rows1, acc.at[dblk.at[l0 + 1]], add=True)

                @pl.when(jj < 3)
                def _():
                    pltpu.make_async_copy(hs_hbm.at[sblk.at[l0 + 2]], rows0, sem0).wait()

                return c2

            lax.fori_loop(0, 4, inner, 0)
            return carry

        lax.fori_loop(0, NCH // 8, outer, 0)

    pl.run_scoped(with_sems, pltpu.SemaphoreType.DMA, pltpu.SemaphoreType.DMA)
    plsc.subcore_barrier()
    pltpu.sync_copy(acc.at[pl.ds(sid * RPS, RPS)], out_hbm.at[cid].at[pl.ds(sid * RPS, RPS)])


def _mm_body(deg_ref, x_ref, w_ref, hs_ref, dinv_ref):
    total = jnp.sum(deg_ref[...], axis=1, keepdims=True) + 1.0
    dinv = lax.rsqrt(total)
    h = jnp.dot(x_ref[...], w_ref[...], preferred_element_type=jnp.float32)
    hs_ref[...] = h * dinv
    dinv_ref[...] = dinv


def _tc_matmul(deg_t, x_pad, W1):
    return pl.pallas_call(
        _mm_body,
        grid=(NBLK,),
        in_specs=[
            pl.BlockSpec((BLK, NW), lambda i: (i, 0)),
            pl.BlockSpec((BLK, D), lambda i: (i, 0)),
            pl.BlockSpec((D, D), lambda i: (0, 0)),
        ],
        out_specs=[
            pl.BlockSpec((BLK, D), lambda i: (i, 0)),
            pl.BlockSpec((BLK, 1), lambda i: (i, 0)),
        ],
        out_shape=[
            jax.ShapeDtypeStruct((NP, D), jnp.float32),
            jax.ShapeDtypeStruct((NP, 1), jnp.float32),
        ],
    )(deg_t, x_pad, W1)


def _head_body(accA, accB, hs, dinv, brow, b1r, fcw, fcbr, out, sums, counts):
    i = pl.program_id(0)

    @pl.when(i == 0)
    def _():
        sums[...] = jnp.zeros_like(sums)
        counts[...] = jnp.zeros_like(counts)

    r = dinv[...] * (accA[...] + accB[...] + hs[...]) + b1r[...]
    r = jnp.maximum(r, 0.0)
    iota = lax.broadcasted_iota(jnp.int32, (G, BLK), 0)
    oh = (jnp.broadcast_to(brow[...], (G, BLK)) == iota).astype(jnp.float32)
    sums[...] += jnp.dot(oh, r, preferred_element_type=jnp.float32)
    counts[...] += jnp.sum(oh, axis=1, keepdims=True)

    @pl.when(i == NBLK - 1)
    def _():
        g = sums[...] / jnp.maximum(counts[...], 1.0)
        logits = jnp.dot(g, fcw[...], preferred_element_type=jnp.float32) + fcbr[...]
        m = jnp.max(logits, axis=1, keepdims=True)
        lse = m + jnp.log(jnp.sum(jnp.exp(logits - m), axis=1, keepdims=True))
        out[...] = logits - lse


def _tc_head(accA, accB, hs, dinv, batch_row, b1r, fcW, fcbr):
    return pl.pallas_call(
        _head_body,
        grid=(NBLK,),
        in_specs=[
            pl.BlockSpec((BLK, D), lambda i: (i, 0)),
            pl.BlockSpec((BLK, D), lambda i: (i, 0)),
            pl.BlockSpec((BLK, D), lambda i: (i, 0)),
            pl.BlockSpec((BLK, 1), lambda i: (i, 0)),
            pl.BlockSpec((1, BLK), lambda i: (0, i)),
            pl.BlockSpec((1, D), lambda i: (0, 0)),
            pl.BlockSpec((D, 2), lambda i: (0, 0)),
            pl.BlockSpec((1, 2), lambda i: (0, 0)),
        ],
        out_specs=pl.BlockSpec((G, 2), lambda i: (0, 0)),
        out_shape=jax.ShapeDtypeStruct((G, 2), jnp.float32),
        scratch_shapes=[
            pltpu.VMEM((G, D), jnp.float32),
            pltpu.VMEM((G, 1), jnp.float32),
        ],
        compiler_params=pltpu.CompilerParams(
            dimension_semantics=("arbitrary",),
        ),
    )(accA, accB, hs, dinv, batch_row, b1r, fcW, fcbr)


def kernel(x, edge_index, batch, W1, b1, fcW, fcb):
    src = edge_index[0].astype(jnp.int32)
    dst = edge_index[1].astype(jnp.int32)
    dst_w = dst.reshape(NW, EPW)
    src_ch = src.reshape(NW, NCH // 8, 8, C)
    dst_ch = dst.reshape(NW, NCH // 8, 8, C)
    x_pad = jnp.pad(x, ((0, NP - N_NODES), (0, 0)))
    batch_row = jnp.pad(
        batch.astype(jnp.int32), (0, NP - N_NODES), constant_values=G
    ).reshape(1, NP)
    zeros_nd = jnp.zeros((RPS, D), jnp.float32)

    deg_part = _sc_degree(dst_w)
    deg_t = deg_part.reshape(NW, NP).T
    hs, dinv = _tc_matmul(deg_t, x_pad, W1)
    acc = _sc_edge_agg(hs, src_ch, dst_ch, zeros_nd)
    out = _tc_head(
        acc[0], acc[1], hs, dinv, batch_row,
        b1.reshape(1, D), fcW, fcb.reshape(1, 2),
    )
    return out


# speedup vs baseline: 1.2089x; 1.0350x over previous
import functools

import jax
import jax.numpy as jnp
from jax import lax
from jax.experimental import pallas as pl
from jax.experimental.pallas import tpu as pltpu
from jax.experimental.pallas import tpu_sc as plsc

N_NODES = 10000
NP = 10240
E = 320000
D = 128
G = 64
NC = 2
NS = 16
NW = NC * NS
EPW = E // NW
C = 125
NCH = EPW // C
RPS = NP // NS
BLK = 1024
NBLK = NP // BLK

_mesh = plsc.VectorSubcoreMesh(core_axis_name="c", subcore_axis_name="s")


@functools.partial(
    pl.kernel,
    out_type=jax.ShapeDtypeStruct((NW, NP // 16, 16), jnp.float32),
    mesh=_mesh,
    scratch_types=[
        pltpu.VMEM((EPW,), jnp.int32),
        pltpu.VMEM((NP // 16, 16), jnp.float32),
    ],
    compiler_params=pltpu.CompilerParams(needs_layout_passes=False),
)
def _sc_degree(dst_hbm, out_hbm, didx, hist):
    cid = lax.axis_index("c")
    sid = lax.axis_index("s")
    wid = cid * NS + sid
    pltpu.sync_copy(dst_hbm.at[wid], didx)

    zeros = jnp.zeros((16,), jnp.float32)

    def zinit(i, carry):
        hist[i, :] = zeros
        return carry

    lax.fori_loop(0, NP // 16, zinit, 0)

    ones = jnp.ones((16,), jnp.float32)

    def body(i, carry):
        idx = didx[pl.ds(i * 16, 16)]
        plsc.addupdate_scatter(hist, [idx >> 4, idx & 15], ones)
        return carry

    lax.fori_loop(0, EPW // 16, body, 0)
    pltpu.sync_copy(hist, out_hbm.at[wid])


@functools.partial(
    pl.kernel,
    out_type=jax.ShapeDtypeStruct((NC, NP, D), jnp.float32),
    mesh=_mesh,
    scratch_types=[
        pltpu.VMEM((16, C), jnp.int32),
        pltpu.VMEM((16, C), jnp.int32),
        pltpu.VMEM((C, D), jnp.float32),
        pltpu.VMEM((C, D), jnp.float32),
        pltpu.VMEM_SHARED((NP, D), jnp.float32),
    ],
    compiler_params=pltpu.CompilerParams(needs_layout_passes=False),
)
def _sc_edge_agg(hs_hbm, src_hbm, dst_hbm, zeros_hbm, out_hbm,
                 sblk, dblk, rows0, rows1, acc):
    cid = lax.axis_index("c")
    sid = lax.axis_index("s")
    wid = cid * NS + sid
    pltpu.sync_copy(zeros_hbm, acc.at[pl.ds(sid * RPS, RPS)])
    plsc.subcore_barrier()

    def with_sems(sem0, sem1):
        def outer(b, carry):
            pltpu.sync_copy(src_hbm.at[wid].at[b], sblk)
            pltpu.sync_copy(dst_hbm.at[wid].at[b], dblk)
            pltpu.sync_copy(hs_hbm.at[sblk.at[0]], rows0)

            def inner(jj, c2):
                l0 = jj * 2
                pltpu.async_copy(hs_hbm.at[sblk.at[l0 + 1]], rows1, sem1)
                pltpu.sync_copy(rows0, acc.at[dblk.at[l0]], add=True)
                pltpu.make_async_copy(hs_hbm.at[sblk.at[l0 + 1]], rows1, sem1).wait()

                @pl.when(jj < 7)
                def _():
                    pltpu.async_copy(hs_hbm.at[sblk.at[l0 + 2]], rows0, sem0)

                pltpu.sync_copy(rows1, acc.at[dblk.at[l0 + 1]], add=True)

                @pl.when(jj < 7)
                def _():
                    pltpu.make_async_copy(hs_hbm.at[sblk.at[l0 + 2]], rows0, sem0).wait()

                return c2

            lax.fori_loop(0, 8, inner, 0)
            return carry

        lax.fori_loop(0, NCH // 16, outer, 0)

    pl.run_scoped(with_sems, pltpu.SemaphoreType.DMA, pltpu.SemaphoreType.DMA)
    plsc.subcore_barrier()
    pltpu.sync_copy(acc.at[pl.ds(sid * RPS, RPS)], out_hbm.at[cid].at[pl.ds(sid * RPS, RPS)])


def _mm_body(deg_ref, x_ref, w_ref, hs_ref, dinv_ref):
    total = jnp.sum(deg_ref[...], axis=1, keepdims=True) + 1.0
    dinv = lax.rsqrt(total)
    h = jnp.dot(x_ref[...], w_ref[...], preferred_element_type=jnp.float32)
    hs_ref[...] = h * dinv
    dinv_ref[...] = dinv


def _tc_matmul(deg_t, x_pad, W1):
    return pl.pallas_call(
        _mm_body,
        grid=(NBLK,),
        in_specs=[
            pl.BlockSpec((BLK, NW), lambda i: (i, 0)),
            pl.BlockSpec((BLK, D), lambda i: (i, 0)),
            pl.BlockSpec((D, D), lambda i: (0, 0)),
        ],
        out_specs=[
            pl.BlockSpec((BLK, D), lambda i: (i, 0)),
            pl.BlockSpec((BLK, 1), lambda i: (i, 0)),
        ],
        out_shape=[
            jax.ShapeDtypeStruct((NP, D), jnp.float32),
            jax.ShapeDtypeStruct((NP, 1), jnp.float32),
        ],
    )(deg_t, x_pad, W1)


def _head_body(accA, accB, hs, dinv, brow, b1r, fcw, fcbr, out, sums, counts):
    i = pl.program_id(0)

    @pl.when(i == 0)
    def _():
        sums[...] = jnp.zeros_like(sums)
        counts[...] = jnp.zeros_like(counts)

    r = dinv[...] * (accA[...] + accB[...] + hs[...]) + b1r[...]
    r = jnp.maximum(r, 0.0)
    iota = lax.broadcasted_iota(jnp.int32, (G, BLK), 0)
    oh = (jnp.broadcast_to(brow[...], (G, BLK)) == iota).astype(jnp.float32)
    sums[...] += jnp.dot(oh, r, preferred_element_type=jnp.float32)
    counts[...] += jnp.sum(oh, axis=1, keepdims=True)

    @pl.when(i == NBLK - 1)
    def _():
        g = sums[...] / jnp.maximum(counts[...], 1.0)
        logits = jnp.dot(g, fcw[...], preferred_element_type=jnp.float32) + fcbr[...]
        m = jnp.max(logits, axis=1, keepdims=True)
        lse = m + jnp.log(jnp.sum(jnp.exp(logits - m), axis=1, keepdims=True))
        out[...] = logits - lse


def _tc_head(accA, accB, hs, dinv, batch_row, b1r, fcW, fcbr):
    return pl.pallas_call(
        _head_body,
        grid=(NBLK,),
        in_specs=[
            pl.BlockSpec((BLK, D), lambda i: (i, 0)),
            pl.BlockSpec((BLK, D), lambda i: (i, 0)),
            pl.BlockSpec((BLK, D), lambda i: (i, 0)),
            pl.BlockSpec((BLK, 1), lambda i: (i, 0)),
            pl.BlockSpec((1, BLK), lambda i: (0, i)),
            pl.BlockSpec((1, D), lambda i: (0, 0)),
            pl.BlockSpec((D, 2), lambda i: (0, 0)),
            pl.BlockSpec((1, 2), lambda i: (0, 0)),
        ],
        out_specs=pl.BlockSpec((G, 2), lambda i: (0, 0)),
        out_shape=jax.ShapeDtypeStruct((G, 2), jnp.float32),
        scratch_shapes=[
            pltpu.VMEM((G, D), jnp.float32),
            pltpu.VMEM((G, 1), jnp.float32),
        ],
        compiler_params=pltpu.CompilerParams(
            dimension_semantics=("arbitrary",),
        ),
    )(accA, accB, hs, dinv, batch_row, b1r, fcW, fcbr)


def kernel(x, edge_index, batch, W1, b1, fcW, fcb):
    src = edge_index[0].astype(jnp.int32)
    dst = edge_index[1].astype(jnp.int32)
    dst_w = dst.reshape(NW, EPW)
    src_ch = src.reshape(NW, NCH // 16, 16, C)
    dst_ch = dst.reshape(NW, NCH // 16, 16, C)
    x_pad = jnp.pad(x, ((0, NP - N_NODES), (0, 0)))
    batch_row = jnp.pad(
        batch.astype(jnp.int32), (0, NP - N_NODES), constant_values=G
    ).reshape(1, NP)
    zeros_nd = jnp.zeros((RPS, D), jnp.float32)

    deg_part = _sc_degree(dst_w)
    deg_t = deg_part.reshape(NW, NP).T
    hs, dinv = _tc_matmul(deg_t, x_pad, W1)
    acc = _sc_edge_agg(hs, src_ch, dst_ch, zeros_nd)
    out = _tc_head(
        acc[0], acc[1], hs, dinv, batch_row,
        b1.reshape(1, D), fcW, fcb.reshape(1, 2),
    )
    return out
